# Initial kernel scaffold; baseline (speedup 1.0000x reference)
import jax
import jax.numpy as jnp
from jax.experimental import pallas as pl


def kernel(x, Wq, Wk, Wv, Wo, bo):
    raise NotImplementedError("write your pallas kernel here")


# speedup vs baseline: 2.3204x; 2.3204x over previous
import functools

import jax
import jax.numpy as jnp
from jax.experimental import pallas as pl
from jax.experimental.pallas import tpu as pltpu

C = 256
CK = C + 128


def _elu1(y):
    return jnp.where(y > 0, y + 1.0, jnp.exp(y))


def _reduce_body(x_ref, wkT_ref, wvT_ref, woT_ref, acc_ref, b_ref, *, steps):
    j = pl.program_id(0)
    xb = x_ref[...]
    k = _elu1(jnp.dot(xb, wkT_ref[...], preferred_element_type=jnp.float32))
    v = jnp.dot(xb, wvT_ref[...], preferred_element_type=jnp.float32)
    v_ext = jnp.concatenate([v, jnp.ones((xb.shape[0], 128), jnp.float32)], axis=1)
    contrib = jax.lax.dot_general(
        k, v_ext, (((0,), (0,)), ((), ())), preferred_element_type=jnp.float32
    )

    @pl.when(j == 0)
    def _():
        acc_ref[...] = contrib

    @pl.when(j != 0)
    def _():
        acc_ref[...] += contrib

    @pl.when(j == steps - 1)
    def _():
        kvs = acc_ref[...]
        m = jnp.dot(kvs[:, :C], woT_ref[...], preferred_element_type=jnp.float32)
        b_ref[...] = jnp.concatenate([m, kvs[:, C:]], axis=1)


def _map_body(x_ref, wqT_ref, b_ref, bo_ref, o_ref):
    xb = x_ref[...]
    q = _elu1(jnp.dot(xb, wqT_ref[...], preferred_element_type=jnp.float32))
    g = jnp.dot(q, b_ref[...], preferred_element_type=jnp.float32)
    z = 1.0 / jnp.maximum(g[:, C:], 1e-6)
    z256 = jnp.concatenate([z, z], axis=1)
    o_ref[...] = g[:, :C] * z256 + bo_ref[...]


@functools.partial(jax.jit, static_argnames=("interpret",))
def kernel(x, Wq, Wk, Wv, Wo, bo, interpret=False):
    n = x.shape[0]
    bn = 4096
    bm = 4096
    s = n // bn
    p = n // bm

    wqT = Wq.T
    wkT = Wk.T
    wvT = Wv.T
    woT = Wo.T
    bo2 = bo.reshape(1, C)

    _, bmat = pl.pallas_call(
        functools.partial(_reduce_body, steps=s),
        grid=(s,),
        in_specs=[
            pl.BlockSpec((bn, C), lambda j: (j, 0)),
            pl.BlockSpec((C, C), lambda j: (0, 0)),
            pl.BlockSpec((C, C), lambda j: (0, 0)),
            pl.BlockSpec((C, C), lambda j: (0, 0)),
        ],
        out_specs=[
            pl.BlockSpec((C, CK), lambda j: (0, 0)),
            pl.BlockSpec((C, CK), lambda j: (0, 0)),
        ],
        out_shape=[
            jax.ShapeDtypeStruct((C, CK), jnp.float32),
            jax.ShapeDtypeStruct((C, CK), jnp.float32),
        ],
        compiler_params=pltpu.CompilerParams(
            dimension_semantics=("arbitrary",),
            vmem_limit_bytes=50 * 1024 * 1024,
        ),
        name="lga_reduce",
        interpret=interpret,
    )(x, wkT, wvT, woT)

    out = pl.pallas_call(
        _map_body,
        grid=(p,),
        in_specs=[
            pl.BlockSpec((bm, C), lambda j: (j, 0)),
            pl.BlockSpec((C, C), lambda j: (0, 0)),
            pl.BlockSpec((C, CK), lambda j: (0, 0)),
            pl.BlockSpec((1, C), lambda j: (0, 0)),
        ],
        out_specs=pl.BlockSpec((bm, C), lambda j: (j, 0)),
        out_shape=jax.ShapeDtypeStruct((n, C), jnp.float32),
        compiler_params=pltpu.CompilerParams(
            dimension_semantics=("parallel",),
            vmem_limit_bytes=50 * 1024 * 1024,
        ),
        name="lga_map",
        interpret=interpret,
    )(x, wqT, bmat, bo2)

    return out


# speedup vs baseline: 2.4637x; 1.0618x over previous
import functools

import jax
import jax.numpy as jnp
from jax.experimental import pallas as pl
from jax.experimental.pallas import tpu as pltpu

C = 256
CK = C + 128


def _elu1(y):
    return jnp.where(y > 0, y + 1.0, jnp.exp(y))


def _reduce_body(x_ref, wkT_ref, wvT_ref, woT_ref, acc_ref, b_ref, *, steps):
    j = pl.program_id(0)
    xb = x_ref[...]
    k = _elu1(jnp.dot(xb, wkT_ref[...], preferred_element_type=jnp.float32))
    v = jnp.dot(xb, wvT_ref[...], preferred_element_type=jnp.float32)
    v_ext = jnp.concatenate([v, jnp.ones((xb.shape[0], 128), jnp.float32)], axis=1)
    contrib = jax.lax.dot_general(
        k, v_ext, (((0,), (0,)), ((), ())), preferred_element_type=jnp.float32
    )

    @pl.when(j == 0)
    def _():
        acc_ref[...] = contrib

    @pl.when(j != 0)
    def _():
        acc_ref[...] += contrib

    @pl.when(j == steps - 1)
    def _():
        kvs = acc_ref[...]
        m = jnp.dot(kvs[:, :C], woT_ref[...], preferred_element_type=jnp.float32)
        b_ref[...] = jnp.concatenate([m, kvs[:, C:]], axis=1)


def _map_body(x_ref, wqT_ref, b_ref, bo_ref, o_ref):
    xb = x_ref[...]
    q = _elu1(jnp.dot(xb, wqT_ref[...], preferred_element_type=jnp.float32))
    g = jnp.dot(q, b_ref[...], preferred_element_type=jnp.float32)
    z = 1.0 / jnp.maximum(g[:, C:], 1e-6)
    z256 = jnp.concatenate([z, z], axis=1)
    o_ref[...] = g[:, :C] * z256 + bo_ref[...]


@functools.partial(jax.jit, static_argnames=("interpret",))
def kernel(x, Wq, Wk, Wv, Wo, bo, interpret=False):
    n = x.shape[0]
    bn = 8192
    bm = 8192
    s = n // bn
    p = n // bm

    wqT = Wq.T
    wkT = Wk.T
    wvT = Wv.T
    woT = Wo.T
    bo2 = bo.reshape(1, C)

    _, bmat = pl.pallas_call(
        functools.partial(_reduce_body, steps=s),
        grid=(s,),
        in_specs=[
            pl.BlockSpec((bn, C), lambda j: (j, 0)),
            pl.BlockSpec((C, C), lambda j: (0, 0)),
            pl.BlockSpec((C, C), lambda j: (0, 0)),
            pl.BlockSpec((C, C), lambda j: (0, 0)),
        ],
        out_specs=[
            pl.BlockSpec((C, CK), lambda j: (0, 0)),
            pl.BlockSpec((C, CK), lambda j: (0, 0)),
        ],
        out_shape=[
            jax.ShapeDtypeStruct((C, CK), jnp.float32),
            jax.ShapeDtypeStruct((C, CK), jnp.float32),
        ],
        compiler_params=pltpu.CompilerParams(
            dimension_semantics=("arbitrary",),
            vmem_limit_bytes=56 * 1024 * 1024,
        ),
        name="lga_reduce",
        interpret=interpret,
    )(x, wkT, wvT, woT)

    out = pl.pallas_call(
        _map_body,
        grid=(p,),
        in_specs=[
            pl.BlockSpec((bm, C), lambda j: (j, 0)),
            pl.BlockSpec((C, C), lambda j: (0, 0)),
            pl.BlockSpec((C, CK), lambda j: (0, 0)),
            pl.BlockSpec((1, C), lambda j: (0, 0)),
        ],
        out_specs=pl.BlockSpec((bm, C), lambda j: (j, 0)),
        out_shape=jax.ShapeDtypeStruct((n, C), jnp.float32),
        compiler_params=pltpu.CompilerParams(
            dimension_semantics=("parallel",),
            vmem_limit_bytes=56 * 1024 * 1024,
        ),
        name="lga_map",
        interpret=interpret,
    )(x, wqT, bmat, bo2)

    return out
